# baseline (speedup 1.0000x reference)
import functools

import jax
import jax.numpy as jnp
from jax import lax
from jax.experimental import pallas as pl
from jax.experimental.pallas import tpu as pltpu
from jax.experimental.pallas import tpu_sc as plsc

_NC = 2
_NS = 16
_NW = _NC * _NS
_L = 16
_B = 16384
_BPW = _B // _NW
_CH = 128
_NCH = _BPW // _CH
_D = 20


def _gather_body(mid_hbm, uid_hbm, movie_hbm, user_hbm,
                 mlo_out, mhi_out, ulo_out, uhi_out,
                 midx, uidx, klo_m, khi_m, klo_u, khi_u,
                 mlo, mhi, ulo, uhi, sem_m, sem_u):
    c = lax.axis_index("c")
    s = lax.axis_index("s")
    wid = s * _NC + c
    pltpu.sync_copy(mid_hbm.at[wid], midx)
    pltpu.sync_copy(uid_hbm.at[wid], uidx)
    for j in range(_NCH):
        for g in range(_CH // _L):
            sl = pl.ds(g * _L, _L)
            im = midx[j, sl]
            km = lax.shift_right_logical(im * 5, 2)
            klo_m[j, sl] = km
            khi_m[j, sl] = km + 1
            iu = uidx[j, sl]
            ku = lax.shift_right_logical(iu * 5, 2)
            klo_u[j, sl] = ku
            khi_u[j, sl] = ku + 1
    copies = []
    for j in range(_NCH):
        dst = pl.ds(j * _CH, _CH)
        copies.append(pltpu.async_copy(
            movie_hbm.at[klo_m.at[j]], mlo.at[dst], sem_m))
        copies.append(pltpu.async_copy(
            movie_hbm.at[khi_m.at[j]], mhi.at[dst], sem_m))
        copies.append(pltpu.async_copy(
            user_hbm.at[klo_u.at[j]], ulo.at[dst], sem_u))
        copies.append(pltpu.async_copy(
            user_hbm.at[khi_u.at[j]], uhi.at[dst], sem_u))
    for cp in copies:
        cp.wait()
    pltpu.sync_copy(mlo, mlo_out.at[wid])
    pltpu.sync_copy(mhi, mhi_out.at[wid])
    pltpu.sync_copy(ulo, ulo_out.at[wid])
    pltpu.sync_copy(uhi, uhi_out.at[wid])


def _sc_gather(mid, uid, movie_flat, user_flat):
    mesh = plsc.VectorSubcoreMesh(core_axis_name="c", subcore_axis_name="s")
    out16 = jax.ShapeDtypeStruct((_NW, _BPW, _L), jnp.float32)
    fn = functools.partial(
        pl.kernel,
        mesh=mesh,
        out_type=[out16, out16, out16, out16],
        scratch_types=[
            pltpu.VMEM((_NCH, _CH), jnp.int32),
            pltpu.VMEM((_NCH, _CH), jnp.int32),
            pltpu.VMEM((_NCH, _CH), jnp.int32),
            pltpu.VMEM((_NCH, _CH), jnp.int32),
            pltpu.VMEM((_NCH, _CH), jnp.int32),
            pltpu.VMEM((_NCH, _CH), jnp.int32),
            pltpu.VMEM((_BPW, _L), jnp.float32),
            pltpu.VMEM((_BPW, _L), jnp.float32),
            pltpu.VMEM((_BPW, _L), jnp.float32),
            pltpu.VMEM((_BPW, _L), jnp.float32),
            pltpu.SemaphoreType.DMA,
            pltpu.SemaphoreType.DMA,
        ],
        compiler_params=pltpu.CompilerParams(use_tc_tiling_on_sc=False),
    )(_gather_body)
    return fn(mid, uid, movie_flat, user_flat)


_BLK = 2048


def _extract20(lo, hi, idx):
    x32 = jnp.concatenate([lo, hi], axis=1)
    phase = jnp.bitwise_and(idx, 3)
    out = jnp.zeros((lo.shape[0], _D), jnp.float32)
    for k in range(4):
        sel = (phase == k).astype(jnp.float32)
        out = out + sel * x32[:, 4 * k:4 * k + _D]
    return out


def _mlp_body(mlo_ref, mhi_ref, ulo_ref, uhi_ref, mid_ref, uid_ref,
              g_ref, v_ref, wgt_ref, bg_ref, w1mt_ref, w1gt_ref, w1ut_ref,
              w1v_ref, b1_ref, w2t_ref, b2_ref, w3t_ref, b3_ref, o_ref):
    m = _extract20(mlo_ref[...], mhi_ref[...], mid_ref[...])
    u = _extract20(ulo_ref[...], uhi_ref[...], uid_ref[...])
    g = jnp.maximum(g_ref[...] @ wgt_ref[...] + bg_ref[...], 0.0)
    h = (m @ w1mt_ref[...] + g @ w1gt_ref[...]
         + u @ w1ut_ref[...] + v_ref[...] * w1v_ref[...]
         + b1_ref[...])
    h = jnp.maximum(h, 0.0)
    h2 = jnp.maximum(h @ w2t_ref[...] + b2_ref[...], 0.0)
    o = h2 @ w3t_ref[...] + b3_ref[...]
    o_ref[...] = jax.nn.sigmoid(o)


def _mlp(mlo, mhi, ulo, uhi, mid, uid, genre, vote, wgt, bg2, w1mt, w1gt,
         w1ut, w1v, b12, w2t, b22, w3t, b32):
    nblk = _B // _BLK
    row_spec = lambda d: pl.BlockSpec((_BLK, d), lambda i: (i, 0))
    full_spec = lambda a: pl.BlockSpec(a.shape, lambda i: (0, 0))
    return pl.pallas_call(
        _mlp_body,
        grid=(nblk,),
        in_specs=[
            row_spec(_L), row_spec(_L), row_spec(_L), row_spec(_L),
            row_spec(1), row_spec(1), row_spec(20), row_spec(1),
            full_spec(wgt), full_spec(bg2), full_spec(w1mt), full_spec(w1gt),
            full_spec(w1ut), full_spec(w1v), full_spec(b12), full_spec(w2t),
            full_spec(b22), full_spec(w3t), full_spec(b32),
        ],
        out_specs=pl.BlockSpec((_BLK, 1), lambda i: (i, 0)),
        out_shape=jax.ShapeDtypeStruct((_B, 1), jnp.float32),
    )(mlo, mhi, ulo, uhi, mid, uid, genre, vote, wgt, bg2, w1mt, w1gt,
      w1ut, w1v, b12, w2t, b22, w3t, b32)


def kernel(userId, movieId, genre, vote_average, release_date, movie_table,
           user_table, Wg, bg, W1, b1, W2, b2, W3, b3):
    del release_date
    mid = movieId.astype(jnp.int32)
    uid = userId.astype(jnp.int32)
    mid3 = mid.reshape(_NW, _NCH, _CH)
    uid3 = uid.reshape(_NW, _NCH, _CH)
    nm, nu = movie_table.shape[0], user_table.shape[0]
    movie_flat = movie_table.reshape(nm * _D // _L, _L)
    user_flat = user_table.reshape(nu * _D // _L, _L)
    mlo, mhi, ulo, uhi = _sc_gather(mid3, uid3, movie_flat, user_flat)
    mlo = mlo.reshape(_B, _L)
    mhi = mhi.reshape(_B, _L)
    ulo = ulo.reshape(_B, _L)
    uhi = uhi.reshape(_B, _L)
    wgt = Wg.T
    w1mt = W1[:, 0:20].T
    w1gt = W1[:, 20:36].T
    w1ut = W1[:, 36:56].T
    w1v = W1[:, 56:57].T
    w2t = W2.T
    w3t = W3.T
    return _mlp(mlo, mhi, ulo, uhi, mid.reshape(_B, 1), uid.reshape(_B, 1),
                genre, vote_average, wgt, bg.reshape(1, 16),
                w1mt, w1gt, w1ut, w1v, b1.reshape(1, 128),
                w2t, b2.reshape(1, 32), w3t, b3.reshape(1, 1))
